# baseline (speedup 1.0000x reference)
import jax
import jax.numpy as jnp
from jax.experimental import pallas as pl

_B, _N, _D = 4, 4096, 256
_S, _K, _ITERS, _H = 64, 16, 3, 128
_BN = _B * _N
_BS = _B * _S
_CH = 2048
_NC = _BN // _CH
_PREC = jax.lax.Precision.HIGHEST


def _ln(x, g, b, eps=1e-5):
    m = jnp.mean(x, axis=-1, keepdims=True)
    xc = x - m
    v = jnp.mean(xc * xc, axis=-1, keepdims=True)
    return xc / jnp.sqrt(v + eps) * g + b


def _dot_nt(a, b):
    return jax.lax.dot_general(a, b, (((1,), (1,)), ((), ())), precision=_PREC)


def _dot_nn(a, b):
    return jax.lax.dot_general(a, b, (((1,), (0,)), ((), ())), precision=_PREC)


def _proj_body(x_ref, Wv_ref, bv_ref, Wk_ref, bk_ref, g_ref, b_ref,
               v_ref, k_ref):
    xn = _ln(x_ref[...], g_ref[...], b_ref[...])
    v_ref[...] = _dot_nt(xn, Wv_ref[...]) + bv_ref[...]
    wkc = jnp.sum(Wk_ref[...], axis=0, keepdims=True)
    k_ref[0] = _dot_nt(wkc, xn) + jnp.sum(bk_ref[...])


def _k1(inp2, Wv, bv, Wk, bk, g, b):
    return pl.pallas_call(
        _proj_body,
        grid=(_NC,),
        in_specs=[
            pl.BlockSpec((_CH, _D), lambda c: (c, 0)),
            pl.BlockSpec((_D, _D), lambda c: (0, 0)),
            pl.BlockSpec((1, _D), lambda c: (0, 0)),
            pl.BlockSpec((_D, _D), lambda c: (0, 0)),
            pl.BlockSpec((1, _D), lambda c: (0, 0)),
            pl.BlockSpec((1, _D), lambda c: (0, 0)),
            pl.BlockSpec((1, _D), lambda c: (0, 0)),
        ],
        out_specs=[
            pl.BlockSpec((_CH, _D), lambda c: (c, 0)),
            pl.BlockSpec((1, 1, _CH), lambda c: (c, 0, 0)),
        ],
        out_shape=[
            jax.ShapeDtypeStruct((_BN, _D), jnp.float32),
            jax.ShapeDtypeStruct((_NC, 1, _CH), jnp.float32),
        ],
    )(inp2, Wv, bv, Wk, bk, g, b)


def _fps_topk_body(pos3_ref, x_ref, ksum_ref, pe_W1T_ref, pe_b1_ref,
                   pe_W2_ref, pe_b2_ref, g_ref, b_ref,
                   slots0_ref, spos_ref, sel_ref, ksn_ref, pes_ref,
                   fars_ref):
    colN = jax.lax.broadcasted_iota(jnp.int32, (_B, _N), 1)
    colS = jax.lax.broadcasted_iota(jnp.int32, (_S, _N), 1)
    colF = jax.lax.broadcasted_iota(jnp.int32, (_S, 128), 1)
    rowF = jax.lax.broadcasted_iota(jnp.int32, (_S, 128), 0)
    px = jnp.concatenate([pos3_ref[b, 0:1, :] for b in range(_B)], axis=0)
    py = jnp.concatenate([pos3_ref[b, 1:2, :] for b in range(_B)], axis=0)
    pz = jnp.concatenate([pos3_ref[b, 2:3, :] for b in range(_B)], axis=0)

    fars_ref[...] = jnp.zeros((_S, 128), jnp.float32)

    def fps_body(t, carry):
        dist, far = carry
        rowt = rowF == t
        upd = jnp.zeros((_S, 128), jnp.float32)
        for b in range(_B):
            fb = far[b:b + 1, 0:1].astype(jnp.float32)
            upd = upd + jnp.where(rowt & (colF == b), fb, 0.0)
        fars_ref[...] += upd
        selN = colN == far
        cx = jnp.sum(jnp.where(selN, px, 0.0), axis=1, keepdims=True)
        cy = jnp.sum(jnp.where(selN, py, 0.0), axis=1, keepdims=True)
        cz = jnp.sum(jnp.where(selN, pz, 0.0), axis=1, keepdims=True)
        d = (px - cx) ** 2 + (py - cy) ** 2 + (pz - cz) ** 2
        dist = jnp.minimum(dist, d)
        m = jnp.max(dist, axis=1, keepdims=True)
        far = jnp.min(jnp.where(dist == m, colN, _N), axis=1, keepdims=True)
        return dist, far.astype(jnp.int32)

    dist0 = jnp.full((_B, _N), 1e10, jnp.float32)
    far0 = jnp.zeros((_B, 1), jnp.int32)
    jax.lax.fori_loop(0, _S, fps_body, (dist0, far0))

    w1x = pe_W1T_ref[0:1, :]
    w1y = pe_W1T_ref[1:2, :]
    w1z = pe_W1T_ref[2:3, :]
    pe_b1 = pe_b1_ref[...]
    pe_c = jnp.sum(pe_W2_ref[...], axis=0, keepdims=True)
    pe_const = jnp.sum(pe_b2_ref[...])

    for b in range(_B):
        sl = slice(b * _S, (b + 1) * _S)
        idx = fars_ref[:, b:b + 1].astype(jnp.int32)
        ohb = (colS == idx).astype(jnp.float32)
        spos_b = _dot_nt(ohb, pos3_ref[b])
        spos_ref[sl, :] = spos_b
        raw = _dot_nn(ohb, x_ref[b * _N:(b + 1) * _N, :])
        slots0_ref[sl, :] = _ln(raw, g_ref[...], b_ref[...])

        pxb = pos3_ref[b, 0:1, :]
        pyb = pos3_ref[b, 1:2, :]
        pzb = pos3_ref[b, 2:3, :]
        spx = spos_b[:, 0:1]
        spy = spos_b[:, 1:2]
        spz = spos_b[:, 2:3]
        work = (spx - pxb) ** 2 + (spy - pyb) ** 2 + (spz - pzb) ** 2
        tab = jnp.concatenate([pxb, pyb, pzb, ksum_ref[b]], axis=0)

        for r in range(_K):
            mn = jnp.min(work, axis=1, keepdims=True)
            sel = jnp.min(jnp.where(work == mn, colS, _N), axis=1, keepdims=True)
            ohr = (colS == sel).astype(jnp.float32)
            gf = _dot_nt(ohr, tab)
            work = jnp.where(colS == sel, 1e30, work)
            sel_ref[sl, r:r + 1] = sel
            ksn_ref[sl, r:r + 1] = gf[:, 3:4]
            hr = jax.nn.relu((spx - gf[:, 0:1]) * w1x + (spy - gf[:, 1:2]) * w1y
                             + (spz - gf[:, 2:3]) * w1z + pe_b1)
            pes_ref[sl, r:r + 1] = (jnp.sum(hr * pe_c, axis=1, keepdims=True)
                                    + pe_const)


def _k2(pos3, inp2, ksum3, pe_W1T, pe_b1, pe_W2, pe_b2, g, b):
    from jax.experimental.pallas import tpu as pltpu
    return pl.pallas_call(
        _fps_topk_body,
        out_shape=[
            jax.ShapeDtypeStruct((_BS, _D), jnp.float32),
            jax.ShapeDtypeStruct((_BS, 3), jnp.float32),
            jax.ShapeDtypeStruct((_BS, _K), jnp.int32),
            jax.ShapeDtypeStruct((_BS, _K), jnp.float32),
            jax.ShapeDtypeStruct((_BS, _K), jnp.float32),
        ],
        scratch_shapes=[pltpu.VMEM((_S, 128), jnp.float32)],
    )(pos3, inp2, ksum3.reshape(_B, 1, _N), pe_W1T, pe_b1, pe_W2, pe_b2, g, b)


def _iter_body(slots0_ref, sel_ref, ksn_ref, pes_ref, vfeat_ref,
               Wq_ref, bq_ref,
               gru_Wih_ref, gru_Whh_ref, gru_bih_ref, gru_bhh_ref,
               mlp_W1_ref, mlp_b1_ref, mlp_W2_ref, mlp_b2_ref,
               ln_s_g_ref, ln_s_b_ref, ln_m_g_ref, ln_m_b_ref,
               out_ref):
    slots = slots0_ref[...]
    ksn = ksn_ref[...]
    pesum = pes_ref[...]
    selb = sel_ref[...]
    colS = jax.lax.broadcasted_iota(jnp.int32, (_S, _N), 1)

    ohn = jnp.concatenate(
        [(colS == selb[:, r:r + 1]).astype(jnp.float32) for r in range(_K)],
        axis=0)
    vn = _dot_nn(ohn, vfeat_ref[...])

    wqc = jnp.sum(Wq_ref[...], axis=0, keepdims=True)
    bq_c = jnp.sum(bq_ref[...])
    rowQ = jax.lax.broadcasted_iota(jnp.int32, (_S, _K * _S), 0)
    colQ = jax.lax.broadcasted_iota(jnp.int32, (_S, _K * _S), 1)

    for _ in range(_ITERS):
        slots_prev = slots
        sn = _ln(slots, ln_s_g_ref[...], ln_s_b_ref[...])
        qsum = jnp.sum(sn * wqc, axis=1, keepdims=True) + bq_c

        scores = qsum - ksn + pesum
        smax = jnp.max(scores, axis=1, keepdims=True)
        e = jnp.exp(scores - smax)
        a = e / jnp.sum(e, axis=1, keepdims=True)
        a = a / (jnp.sum(a, axis=0, keepdims=True) + 1e-6)

        arow = jnp.zeros((_S, _K * _S), jnp.float32)
        for r in range(_K):
            arow = arow + jnp.where(colQ == rowQ + r * _S,
                                    a[:, r:r + 1], 0.0)
        upd = _dot_nn(arow, vn)

        gi = _dot_nt(upd, gru_Wih_ref[...]) + gru_bih_ref[...]
        gh = _dot_nt(slots_prev, gru_Whh_ref[...]) + gru_bhh_ref[...]
        i_r = gi[:, :_D]
        i_z = gi[:, _D:2 * _D]
        i_n = gi[:, 2 * _D:]
        h_r = gh[:, :_D]
        h_z = gh[:, _D:2 * _D]
        h_n = gh[:, 2 * _D:]
        r_g = jax.nn.sigmoid(i_r + h_r)
        z_g = jax.nn.sigmoid(i_z + h_z)
        n_g = jnp.tanh(i_n + r_g * h_n)
        slots = (1.0 - z_g) * n_g + z_g * slots_prev

        mid = jax.nn.relu(
            _dot_nt(_ln(slots, ln_m_g_ref[...], ln_m_b_ref[...]), mlp_W1_ref[...])
            + mlp_b1_ref[...])
        slots = slots + _dot_nt(mid, mlp_W2_ref[...]) + mlp_b2_ref[...]

    out_ref[...] = slots


def _k5(slots0, sel, ksn, pes, vfeat, Wq, bq, gru_Wih, gru_Whh, gru_bih,
        gru_bhh, mlp_W1, mlp_b1, mlp_W2, mlp_b2, ln_s_g, ln_s_b,
        ln_m_g, ln_m_b):
    w = lambda shape: pl.BlockSpec(shape, lambda i: (0,) * len(shape))
    return pl.pallas_call(
        _iter_body,
        grid=(_B,),
        in_specs=[
            pl.BlockSpec((_S, _D), lambda i: (i, 0)),
            pl.BlockSpec((_S, _K), lambda i: (i, 0)),
            pl.BlockSpec((_S, _K), lambda i: (i, 0)),
            pl.BlockSpec((_S, _K), lambda i: (i, 0)),
            pl.BlockSpec((_N, _D), lambda i: (i, 0)),
            w((_D, _D)), w((1, _D)),
            w((3 * _D, _D)), w((3 * _D, _D)), w((1, 3 * _D)), w((1, 3 * _D)),
            w((_H, _D)), w((1, _H)), w((_D, _H)), w((1, _D)),
            w((1, _D)), w((1, _D)), w((1, _D)), w((1, _D)),
        ],
        out_specs=pl.BlockSpec((_S, _D), lambda i: (i, 0)),
        out_shape=jax.ShapeDtypeStruct((_BS, _D), jnp.float32),
    )(slots0, sel, ksn, pes, vfeat, Wq, bq, gru_Wih, gru_Whh, gru_bih,
      gru_bhh, mlp_W1, mlp_b1, mlp_W2, mlp_b2, ln_s_g, ln_s_b,
      ln_m_g, ln_m_b)


def kernel(inputs, pos, Wq, bq, Wk, bk, Wv, bv, pe_W1, pe_b1, pe_W2, pe_b2,
           gru_Wih, gru_Whh, gru_bih, gru_bhh, mlp_W1, mlp_b1, mlp_W2, mlp_b2,
           ln_in_g, ln_in_b, ln_s_g, ln_s_b, ln_m_g, ln_m_b):
    inp2 = inputs.reshape(_BN, _D)
    pos3 = jnp.transpose(pos, (0, 2, 1))
    row = lambda v: v.reshape(1, -1)

    vfeat, kt = _k1(inp2, Wv, row(bv), Wk, row(bk), row(ln_in_g), row(ln_in_b))
    slots0, spos, sel, ksn, pes = _k2(
        pos3, inp2, kt, pe_W1.T, row(pe_b1), pe_W2, row(pe_b2),
        row(ln_in_g), row(ln_in_b))
    slots = _k5(slots0, sel, ksn, pes, vfeat, Wq, row(bq),
                gru_Wih, gru_Whh, row(gru_bih), row(gru_bhh),
                mlp_W1, row(mlp_b1), mlp_W2, row(mlp_b2),
                row(ln_s_g), row(ln_s_b), row(ln_m_g), row(ln_m_b))
    return slots.reshape(_B, _S, _D), spos.reshape(_B, _S, 3)


# speedup vs baseline: 1.1604x; 1.1604x over previous
import jax
import jax.numpy as jnp
from jax.experimental import pallas as pl

_B, _N, _D = 4, 4096, 256
_S, _K, _ITERS, _H = 64, 16, 3, 128
_BN = _B * _N
_BS = _B * _S
_CH = 2048
_NC = _BN // _CH
_PREC = jax.lax.Precision.HIGHEST


def _ln(x, g, b, eps=1e-5):
    m = jnp.mean(x, axis=-1, keepdims=True)
    xc = x - m
    v = jnp.mean(xc * xc, axis=-1, keepdims=True)
    return xc / jnp.sqrt(v + eps) * g + b


def _dot_nt(a, b):
    return jax.lax.dot_general(a, b, (((1,), (1,)), ((), ())), precision=_PREC)


def _dot_nn(a, b):
    return jax.lax.dot_general(a, b, (((1,), (0,)), ((), ())), precision=_PREC)


def _proj_body(x_ref, Wv_ref, bv_ref, Wk_ref, bk_ref, g_ref, b_ref,
               v_ref, k_ref):
    xn = _ln(x_ref[...], g_ref[...], b_ref[...])
    v_ref[...] = _dot_nt(xn, Wv_ref[...]) + bv_ref[...]
    wkc = jnp.sum(Wk_ref[...], axis=0, keepdims=True)
    k_ref[0] = _dot_nt(wkc, xn) + jnp.sum(bk_ref[...])


def _k1(inp2, Wv, bv, Wk, bk, g, b):
    return pl.pallas_call(
        _proj_body,
        grid=(_NC,),
        in_specs=[
            pl.BlockSpec((_CH, _D), lambda c: (c, 0)),
            pl.BlockSpec((_D, _D), lambda c: (0, 0)),
            pl.BlockSpec((1, _D), lambda c: (0, 0)),
            pl.BlockSpec((_D, _D), lambda c: (0, 0)),
            pl.BlockSpec((1, _D), lambda c: (0, 0)),
            pl.BlockSpec((1, _D), lambda c: (0, 0)),
            pl.BlockSpec((1, _D), lambda c: (0, 0)),
        ],
        out_specs=[
            pl.BlockSpec((_CH, _D), lambda c: (c, 0)),
            pl.BlockSpec((1, 1, _CH), lambda c: (c, 0, 0)),
        ],
        out_shape=[
            jax.ShapeDtypeStruct((_BN, _D), jnp.float32),
            jax.ShapeDtypeStruct((_NC, 1, _CH), jnp.float32),
        ],
    )(inp2, Wv, bv, Wk, bk, g, b)


def _fps_topk_body(pos3_ref, x_ref, ksum_ref, pe_W1T_ref, pe_b1_ref,
                   pe_W2_ref, pe_b2_ref, g_ref, b_ref,
                   slots0_ref, spos_ref, sel_ref, ksn_ref, pes_ref,
                   fars_ref):
    colN = jax.lax.broadcasted_iota(jnp.int32, (_B, _N), 1)
    colS = jax.lax.broadcasted_iota(jnp.int32, (_S, _N), 1)
    colF = jax.lax.broadcasted_iota(jnp.int32, (_S, 128), 1)
    rowF = jax.lax.broadcasted_iota(jnp.int32, (_S, 128), 0)
    px = jnp.concatenate([pos3_ref[b, 0:1, :] for b in range(_B)], axis=0)
    py = jnp.concatenate([pos3_ref[b, 1:2, :] for b in range(_B)], axis=0)
    pz = jnp.concatenate([pos3_ref[b, 2:3, :] for b in range(_B)], axis=0)

    fars_ref[...] = jnp.zeros((_S, 128), jnp.float32)

    def fps_body(t, carry):
        dist, far = carry
        rowt = rowF == t
        upd = jnp.zeros((_S, 128), jnp.float32)
        for b in range(_B):
            fb = far[b:b + 1, 0:1].astype(jnp.float32)
            upd = upd + jnp.where(rowt & (colF == b), fb, 0.0)
        fars_ref[...] += upd
        selN = colN == far
        cx = jnp.sum(jnp.where(selN, px, 0.0), axis=1, keepdims=True)
        cy = jnp.sum(jnp.where(selN, py, 0.0), axis=1, keepdims=True)
        cz = jnp.sum(jnp.where(selN, pz, 0.0), axis=1, keepdims=True)
        d = (px - cx) ** 2 + (py - cy) ** 2 + (pz - cz) ** 2
        dist = jnp.minimum(dist, d)
        m = jnp.max(dist, axis=1, keepdims=True)
        far = jnp.min(jnp.where(dist == m, colN, _N), axis=1, keepdims=True)
        return dist, far.astype(jnp.int32)

    dist0 = jnp.full((_B, _N), 1e10, jnp.float32)
    far0 = jnp.zeros((_B, 1), jnp.int32)
    jax.lax.fori_loop(0, _S, fps_body, (dist0, far0))

    w1x = pe_W1T_ref[0:1, :]
    w1y = pe_W1T_ref[1:2, :]
    w1z = pe_W1T_ref[2:3, :]
    pe_b1 = pe_b1_ref[...]
    pe_c = jnp.sum(pe_W2_ref[...], axis=0, keepdims=True)
    pe_const = jnp.sum(pe_b2_ref[...])

    for b in range(_B):
        sl = slice(b * _S, (b + 1) * _S)
        idx = fars_ref[:, b:b + 1].astype(jnp.int32)
        ohb = (colS == idx).astype(jnp.float32)
        spos_b = _dot_nt(ohb, pos3_ref[b])
        spos_ref[sl, :] = spos_b
        raw = _dot_nn(ohb, x_ref[b * _N:(b + 1) * _N, :])
        slots0_ref[sl, :] = _ln(raw, g_ref[...], b_ref[...])

        pxb = pos3_ref[b, 0:1, :]
        pyb = pos3_ref[b, 1:2, :]
        pzb = pos3_ref[b, 2:3, :]
        spx = spos_b[:, 0:1]
        spy = spos_b[:, 1:2]
        spz = spos_b[:, 2:3]
        work = (spx - pxb) ** 2 + (spy - pyb) ** 2 + (spz - pzb) ** 2
        tab = jnp.concatenate([pxb, pyb, pzb, ksum_ref[b]], axis=0)

        for r in range(_K):
            mn = jnp.min(work, axis=1, keepdims=True)
            sel = jnp.min(jnp.where(work == mn, colS, _N), axis=1, keepdims=True)
            ohr = (colS == sel).astype(jnp.float32)
            gf = _dot_nt(ohr, tab)
            work = jnp.where(colS == sel, 1e30, work)
            sel_ref[sl, r:r + 1] = sel
            ksn_ref[sl, r:r + 1] = gf[:, 3:4]
            hr = jax.nn.relu((spx - gf[:, 0:1]) * w1x + (spy - gf[:, 1:2]) * w1y
                             + (spz - gf[:, 2:3]) * w1z + pe_b1)
            pes_ref[sl, r:r + 1] = (jnp.sum(hr * pe_c, axis=1, keepdims=True)
                                    + pe_const)


def _k2(pos3, inp2, ksum3, pe_W1T, pe_b1, pe_W2, pe_b2, g, b):
    from jax.experimental.pallas import tpu as pltpu
    return pl.pallas_call(
        _fps_topk_body,
        out_shape=[
            jax.ShapeDtypeStruct((_BS, _D), jnp.float32),
            jax.ShapeDtypeStruct((_BS, 3), jnp.float32),
            jax.ShapeDtypeStruct((_BS, _K), jnp.int32),
            jax.ShapeDtypeStruct((_BS, _K), jnp.float32),
            jax.ShapeDtypeStruct((_BS, _K), jnp.float32),
        ],
        scratch_shapes=[pltpu.VMEM((_S, 128), jnp.float32)],
    )(pos3, inp2, ksum3.reshape(_B, 1, _N), pe_W1T, pe_b1, pe_W2, pe_b2, g, b)


def _iter_body(slots0_ref, sel_ref, ksn_ref, pes_ref, vfeat_ref,
               Wq_ref, bq_ref,
               gru_Wih_ref, gru_Whh_ref, gru_bih_ref, gru_bhh_ref,
               mlp_W1_ref, mlp_b1_ref, mlp_W2_ref, mlp_b2_ref,
               ln_s_g_ref, ln_s_b_ref, ln_m_g_ref, ln_m_b_ref,
               out_ref):
    slots = slots0_ref[...]
    ksn = ksn_ref[...]
    pesum = pes_ref[...]
    selb = sel_ref[...]
    colS = jax.lax.broadcasted_iota(jnp.int32, (_S, _N), 1)

    wqc = jnp.sum(Wq_ref[...], axis=0, keepdims=True)
    bq_c = jnp.sum(bq_ref[...])

    for _ in range(_ITERS):
        slots_prev = slots
        sn = _ln(slots, ln_s_g_ref[...], ln_s_b_ref[...])
        qsum = jnp.sum(sn * wqc, axis=1, keepdims=True) + bq_c

        scores = qsum - ksn + pesum
        smax = jnp.max(scores, axis=1, keepdims=True)
        e = jnp.exp(scores - smax)
        a = e / jnp.sum(e, axis=1, keepdims=True)
        a = a / (jnp.sum(a, axis=0, keepdims=True) + 1e-6)

        amat = jnp.zeros((_S, _N), jnp.float32)
        for r in range(_K):
            amat = amat + jnp.where(colS == selb[:, r:r + 1],
                                    a[:, r:r + 1], 0.0)
        upd = _dot_nn(amat, vfeat_ref[...])

        gi = _dot_nt(upd, gru_Wih_ref[...]) + gru_bih_ref[...]
        gh = _dot_nt(slots_prev, gru_Whh_ref[...]) + gru_bhh_ref[...]
        i_r = gi[:, :_D]
        i_z = gi[:, _D:2 * _D]
        i_n = gi[:, 2 * _D:]
        h_r = gh[:, :_D]
        h_z = gh[:, _D:2 * _D]
        h_n = gh[:, 2 * _D:]
        r_g = jax.nn.sigmoid(i_r + h_r)
        z_g = jax.nn.sigmoid(i_z + h_z)
        n_g = jnp.tanh(i_n + r_g * h_n)
        slots = (1.0 - z_g) * n_g + z_g * slots_prev

        mid = jax.nn.relu(
            _dot_nt(_ln(slots, ln_m_g_ref[...], ln_m_b_ref[...]), mlp_W1_ref[...])
            + mlp_b1_ref[...])
        slots = slots + _dot_nt(mid, mlp_W2_ref[...]) + mlp_b2_ref[...]

    out_ref[...] = slots


def _k5(slots0, sel, ksn, pes, vfeat, Wq, bq, gru_Wih, gru_Whh, gru_bih,
        gru_bhh, mlp_W1, mlp_b1, mlp_W2, mlp_b2, ln_s_g, ln_s_b,
        ln_m_g, ln_m_b):
    w = lambda shape: pl.BlockSpec(shape, lambda i: (0,) * len(shape))
    return pl.pallas_call(
        _iter_body,
        grid=(_B,),
        in_specs=[
            pl.BlockSpec((_S, _D), lambda i: (i, 0)),
            pl.BlockSpec((_S, _K), lambda i: (i, 0)),
            pl.BlockSpec((_S, _K), lambda i: (i, 0)),
            pl.BlockSpec((_S, _K), lambda i: (i, 0)),
            pl.BlockSpec((_N, _D), lambda i: (i, 0)),
            w((_D, _D)), w((1, _D)),
            w((3 * _D, _D)), w((3 * _D, _D)), w((1, 3 * _D)), w((1, 3 * _D)),
            w((_H, _D)), w((1, _H)), w((_D, _H)), w((1, _D)),
            w((1, _D)), w((1, _D)), w((1, _D)), w((1, _D)),
        ],
        out_specs=pl.BlockSpec((_S, _D), lambda i: (i, 0)),
        out_shape=jax.ShapeDtypeStruct((_BS, _D), jnp.float32),
    )(slots0, sel, ksn, pes, vfeat, Wq, bq, gru_Wih, gru_Whh, gru_bih,
      gru_bhh, mlp_W1, mlp_b1, mlp_W2, mlp_b2, ln_s_g, ln_s_b,
      ln_m_g, ln_m_b)


def kernel(inputs, pos, Wq, bq, Wk, bk, Wv, bv, pe_W1, pe_b1, pe_W2, pe_b2,
           gru_Wih, gru_Whh, gru_bih, gru_bhh, mlp_W1, mlp_b1, mlp_W2, mlp_b2,
           ln_in_g, ln_in_b, ln_s_g, ln_s_b, ln_m_g, ln_m_b):
    inp2 = inputs.reshape(_BN, _D)
    pos3 = jnp.transpose(pos, (0, 2, 1))
    row = lambda v: v.reshape(1, -1)

    vfeat, kt = _k1(inp2, Wv, row(bv), Wk, row(bk), row(ln_in_g), row(ln_in_b))
    slots0, spos, sel, ksn, pes = _k2(
        pos3, inp2, kt, pe_W1.T, row(pe_b1), pe_W2, row(pe_b2),
        row(ln_in_g), row(ln_in_b))
    slots = _k5(slots0, sel, ksn, pes, vfeat, Wq, row(bq),
                gru_Wih, gru_Whh, row(gru_bih), row(gru_bhh),
                mlp_W1, row(mlp_b1), mlp_W2, row(mlp_b2),
                row(ln_s_g), row(ln_s_b), row(ln_m_g), row(ln_m_b))
    return slots.reshape(_B, _S, _D), spos.reshape(_B, _S, 3)
